# Initial kernel scaffold; baseline (speedup 1.0000x reference)
import jax
import jax.numpy as jnp
from jax.experimental import pallas as pl


def kernel(x, table, gamma, beta, W, b):
    raise NotImplementedError("write your pallas kernel here")


# speedup vs baseline: 180.7876x; 180.7876x over previous
import functools

import jax
import jax.numpy as jnp
from jax import lax
from jax.experimental import pallas as pl
from jax.experimental.pallas import tpu as pltpu
from jax.experimental.pallas import tpu_sc as plsc

NC, NS = 2, 16
NW = NC * NS
LANES = 16


def _rsqrt(v):
    s = v * 0.5 + 0.5
    for _ in range(24):
        s = 0.5 * (s + v / s)
    return 1.0 / s


@functools.lru_cache(maxsize=None)
def _build(B, SEQ):
    rows_per_w = B // NW
    chunk = 128
    nchunk = rows_per_w // chunk

    mesh = plsc.VectorSubcoreMesh(
        core_axis_name="c", subcore_axis_name="s",
        num_cores=NC, num_subcores=NS)

    @functools.partial(
        pl.kernel,
        out_type=jax.ShapeDtypeStruct((B, 2), jnp.float32),
        mesh=mesh,
        scratch_types=[
            pltpu.VMEM((chunk, SEQ), jnp.int32),
            pltpu.VMEM((4, LANES), jnp.float32),
            pltpu.VMEM((32,), jnp.float32),
            pltpu.VMEM((LANES,), jnp.float32),
            pltpu.VMEM((LANES,), jnp.float32),
            pltpu.VMEM((rows_per_w, 2), jnp.float32),
        ],
        compiler_params=pltpu.CompilerParams(use_tc_tiling_on_sc=False,
                                             needs_layout_passes=False),
    )
    def tiny_kernel(x_hbm, tcols_hbm, params_hbm, out_hbm,
                    xv, tcols_v, params_v, proj0_v, proj1_v, outv):
        wid = lax.axis_index("s") * NC + lax.axis_index("c")

        pltpu.sync_copy(tcols_hbm, tcols_v)
        pltpu.sync_copy(params_hbm, params_v)

        pa = params_v[pl.ds(0, LANES)]
        pb = params_v[pl.ds(LANES, LANES)]

        c = [tcols_v[k] for k in range(4)]
        mu = (c[0] + c[1] + c[2] + c[3]) * 0.25
        d = [ck - mu for ck in c]
        var = (d[0] * d[0] + d[1] * d[1] + d[2] * d[2] + d[3] * d[3]) * 0.25
        r = _rsqrt(var + 1e-5)
        ln = [d[k] * r * pa[k] + pa[4 + k] for k in range(4)]
        inv = 1.0 / SEQ
        t0 = (ln[0] * pa[8] + ln[1] * pa[9]
              + ln[2] * pa[10] + ln[3] * pa[11]
              + pb[0]) * inv
        t1 = (ln[0] * pa[12] + ln[1] * pa[13]
              + ln[2] * pa[14] + ln[3] * pa[15]
              + pb[1]) * inv
        proj0_v[...] = t0
        proj1_v[...] = t1

        iota = lax.iota(jnp.int32, LANES)
        zeros = jnp.zeros((LANES,), jnp.float32)

        for ci in range(nchunk):
            base = wid * rows_per_w + ci * chunk
            pltpu.sync_copy(x_hbm.at[pl.ds(base, chunk)], xv)
            for g in range(chunk // LANES):
                rows = g * LANES + iota

                def lbody(l, acc, rows=rows):
                    a0, a1 = acc
                    colv = jnp.full((LANES,), l, jnp.int32)
                    xi = plsc.load_gather(xv, [rows, colv])
                    a0 = a0 + plsc.load_gather(proj0_v, [xi])
                    a1 = a1 + plsc.load_gather(proj1_v, [xi])
                    return a0, a1

                a0, a1 = lax.fori_loop(0, SEQ, lbody, (zeros, zeros),
                                       unroll=8)
                orow = ci * chunk + g * LANES + iota
                plsc.store_scatter(outv, [orow, jnp.zeros((LANES,), jnp.int32)], a0)
                plsc.store_scatter(outv, [orow, jnp.ones((LANES,), jnp.int32)], a1)

        pltpu.sync_copy(outv, out_hbm.at[pl.ds(wid * rows_per_w, rows_per_w)])

    return tiny_kernel


def kernel(x, table, gamma, beta, W, b):
    B, SEQ = x.shape
    tcols = jnp.pad(table.T, ((0, 0), (0, LANES - table.shape[0])))
    params = jnp.concatenate(
        [gamma, beta, W.reshape(-1), b,
         jnp.zeros((32 - 18,), jnp.float32)]).astype(jnp.float32)
    return _build(B, SEQ)(x, tcols, params)
